# baseline (speedup 1.0000x reference)
import functools

import jax
import jax.numpy as jnp
from jax.experimental import pallas as pl

_TEMP = 0.25
_K = 8


def _block_kernel(x_ref, addr_ref, vals_ref, wq_ref, wout_ref, out_ref):
    xb = x_ref[...]
    addr = addr_ref[...]
    vals = vals_ref[...]
    wq = wq_ref[...]
    wout = wout_ref[...]

    qT = jax.lax.dot_general(wq, xb, (((1,), (1,)), ((), ())),
                             preferred_element_type=jnp.float32)
    qn = jnp.sum(qT * qT, axis=0, keepdims=True)
    qT = qT * jax.lax.rsqrt(jnp.maximum(qn, 1e-24))

    an = jnp.sum(addr * addr, axis=-1, keepdims=True)
    addr = addr * jax.lax.rsqrt(jnp.maximum(an, 1e-24))

    sT = jnp.dot(addr, qT, preferred_element_type=jnp.float32)
    sT = sT * (1.0 / _TEMP)

    neg = jnp.float32(-jnp.inf)
    cur = sT
    t = jnp.max(cur, axis=0, keepdims=True)
    smax = t
    for _ in range(_K - 1):
        cur = jnp.where(cur >= t, neg, cur)
        t = jnp.max(cur, axis=0, keepdims=True)

    sel = sT >= t
    e = jnp.where(sel, jnp.exp(sT - smax), 0.0)
    w = e * (1.0 / jnp.sum(e, axis=0, keepdims=True))

    vw = jax.lax.dot_general(vals, wout, (((1,), (1,)), ((), ())),
                             preferred_element_type=jnp.float32)
    out_ref[...] = jax.lax.dot_general(w, vw, (((0,), (0,)), ((), ())),
                                       preferred_element_type=jnp.float32)


@functools.partial(jax.jit, static_argnames=("interpret",))
def kernel(x, memory_addresses, memory_values, W_q, W_out, interpret=False):
    B, C, D = x.shape
    M, A = memory_addresses.shape
    CB = 8192
    x2 = x.reshape(B * C, D)
    mv2 = memory_values.reshape(B * M, D)
    blocks_per_batch = C // CB

    out2 = pl.pallas_call(
        _block_kernel,
        grid=(B * blocks_per_batch,),
        in_specs=[
            pl.BlockSpec((CB, D), lambda i: (i, 0)),
            pl.BlockSpec((M, A), lambda i: (0, 0)),
            pl.BlockSpec((M, D), lambda i: (i // blocks_per_batch, 0)),
            pl.BlockSpec((A, D), lambda i: (0, 0)),
            pl.BlockSpec((D, D), lambda i: (0, 0)),
        ],
        out_specs=pl.BlockSpec((CB, D), lambda i: (i, 0)),
        out_shape=jax.ShapeDtypeStruct((B * C, D), jnp.float32),
        interpret=interpret,
    )(x2, memory_addresses, mv2, W_q, W_out)
    return out2.reshape(B, C, D)


# speedup vs baseline: 3.3878x; 3.3878x over previous
import functools

import jax
import jax.numpy as jnp
from jax.experimental import pallas as pl

_TEMP = 0.25
_K = 8


def _block_kernel(xt_ref, addrt_ref, vals_ref, wq_ref, wout_ref, out_ref):
    xT = xt_ref[0]
    addrT = addrt_ref[...]
    vals = vals_ref[0]
    wq = wq_ref[...]
    wout = wout_ref[...]

    qT = jnp.dot(wq, xT, preferred_element_type=jnp.float32)
    qn = jnp.sum(qT * qT, axis=0, keepdims=True)
    qT = qT * jax.lax.rsqrt(jnp.maximum(qn, 1e-24))

    an = jnp.sum(addrT * addrT, axis=0, keepdims=True)
    addrT = addrT * jax.lax.rsqrt(jnp.maximum(an, 1e-24))

    sT = jax.lax.dot_general(addrT, qT, (((0,), (0,)), ((), ())),
                             preferred_element_type=jnp.float32)
    sT = sT * (1.0 / _TEMP)

    neg = jnp.float32(-jnp.inf)
    cur = sT
    t = jnp.max(cur, axis=0, keepdims=True)
    smax = t
    for _ in range(_K - 1):
        cur = jnp.where(cur >= t, neg, cur)
        t = jnp.max(cur, axis=0, keepdims=True)

    sel = sT >= t
    e = jnp.where(sel, jnp.exp(sT - smax), 0.0)
    w = e * (1.0 / jnp.sum(e, axis=0, keepdims=True))

    vw = jax.lax.dot_general(vals, wout, (((1,), (1,)), ((), ())),
                             preferred_element_type=jnp.float32)
    out_ref[0] = jax.lax.dot_general(vw, w, (((0,), (0,)), ((), ())),
                                     preferred_element_type=jnp.float32)


@functools.partial(jax.jit, static_argnames=("interpret",))
def kernel(x, memory_addresses, memory_values, W_q, W_out, interpret=False):
    B, C, D = x.shape
    M, A = memory_addresses.shape
    CB = 8192

    xt = jnp.transpose(x, (0, 2, 1))
    addrT = jnp.transpose(memory_addresses)

    outT = pl.pallas_call(
        _block_kernel,
        grid=(B, C // CB),
        in_specs=[
            pl.BlockSpec((1, D, CB), lambda b, c: (b, 0, c)),
            pl.BlockSpec((A, M), lambda b, c: (0, 0)),
            pl.BlockSpec((1, M, D), lambda b, c: (b, 0, 0)),
            pl.BlockSpec((A, D), lambda b, c: (0, 0)),
            pl.BlockSpec((D, D), lambda b, c: (0, 0)),
        ],
        out_specs=pl.BlockSpec((1, D, CB), lambda b, c: (b, 0, c)),
        out_shape=jax.ShapeDtypeStruct((B, D, C), jnp.float32),
        interpret=interpret,
    )(xt, addrT, memory_values, W_q, W_out)
    return jnp.transpose(outT, (0, 2, 1))


# speedup vs baseline: 4.2744x; 1.2617x over previous
import functools

import jax
import jax.numpy as jnp
from jax.experimental import pallas as pl

_TEMP = 0.25
_K = 8


def _block_kernel(xt_ref, addrt_ref, vals_ref, wq_ref, wout_ref, out_ref):
    xT = xt_ref[0]
    addrT = addrt_ref[...]
    vals = vals_ref[0]
    wq = wq_ref[...]
    wout = wout_ref[...]

    qT = jnp.dot(wq, xT, preferred_element_type=jnp.float32)
    qn = jnp.sum(qT * qT, axis=0, keepdims=True)
    qT = qT * jax.lax.rsqrt(jnp.maximum(qn, 1e-24))

    an = jnp.sum(addrT * addrT, axis=0, keepdims=True)
    scale = jnp.float32(1.4426950408889634 / _TEMP)
    addrT = addrT * (jax.lax.rsqrt(jnp.maximum(an, 1e-24)) * scale)

    sT = jax.lax.dot_general(addrT, qT, (((0,), (0,)), ((), ())),
                             preferred_element_type=jnp.float32)

    S = [sT[8 * k:8 * (k + 1), :] for k in range(8)]
    _NET = ((0, 1), (2, 3), (4, 5), (6, 7),
            (0, 2), (1, 3), (4, 6), (5, 7),
            (1, 2), (5, 6), (0, 4), (3, 7),
            (1, 5), (2, 6),
            (1, 4), (3, 6),
            (2, 4), (3, 5),
            (3, 4))
    for i, j in _NET:
        hi = jnp.maximum(S[i], S[j])
        lo = jnp.minimum(S[i], S[j])
        S[i], S[j] = hi, lo

    smax = None
    for i in range(_K):
        t = jnp.max(S[0], axis=0, keepdims=True)
        if i == 0:
            smax = t
        if i == _K - 1:
            break
        pop = S[0] == t
        for k in range(_K - 1 - i):
            S[k] = jnp.where(pop, S[k + 1], S[k])

    sel = sT >= t
    e = jnp.where(sel, jnp.exp2(sT - smax), 0.0)
    w = e * (1.0 / jnp.sum(e, axis=0, keepdims=True))

    vw = jax.lax.dot_general(vals, wout, (((1,), (1,)), ((), ())),
                             preferred_element_type=jnp.float32)
    out_ref[0] = jax.lax.dot_general(vw, w, (((0,), (0,)), ((), ())),
                                     preferred_element_type=jnp.float32)


@functools.partial(jax.jit, static_argnames=("interpret",))
def kernel(x, memory_addresses, memory_values, W_q, W_out, interpret=False):
    B, C, D = x.shape
    M, A = memory_addresses.shape
    CB = 8192

    xt = jnp.transpose(x, (0, 2, 1))
    addrT = jnp.transpose(memory_addresses)

    outT = pl.pallas_call(
        _block_kernel,
        grid=(B, C // CB),
        in_specs=[
            pl.BlockSpec((1, D, CB), lambda b, c: (b, 0, c)),
            pl.BlockSpec((A, M), lambda b, c: (0, 0)),
            pl.BlockSpec((1, M, D), lambda b, c: (b, 0, 0)),
            pl.BlockSpec((A, D), lambda b, c: (0, 0)),
            pl.BlockSpec((D, D), lambda b, c: (0, 0)),
        ],
        out_specs=pl.BlockSpec((1, D, CB), lambda b, c: (b, 0, c)),
        out_shape=jax.ShapeDtypeStruct((B, D, C), jnp.float32),
        interpret=interpret,
    )(xt, addrT, memory_values, W_q, W_out)
    return jnp.transpose(outT, (0, 2, 1))
